# baseline (speedup 1.0000x reference)
import jax
import jax.numpy as jnp
from jax import lax
from jax.experimental import pallas as pl
from jax.experimental.pallas import tpu as pltpu
from jax.experimental.pallas import tpu_sc as plsc

CAP = 16384
B = 4096
ROW = 3072
NC, NS, L = 2, 16, 16
NW = NC * NS
RPW = CAP // NW
JPW = B // NW
K = 16
HALF = CAP // 2


def _sc_body_half(lo, with_y):
    def body(*args):
        if with_y:
            (val_hbm, idx_hbm, w_hbm, valy_hbm, idx3_hbm, x_hbm, y_hbm,
             idx_v, w_v, valy_v, myy_v, tgt_v, tgt2_v,
             buf0, buf1, semg0, semg1, sems0, sems1) = args
        else:
            (val_hbm, idx_hbm, w_hbm, valy_hbm, idx3_hbm, x_hbm,
             idx_v, w_v, valy_v, myy_v, tgt_v, tgt2_v,
             buf0, buf1, semg0, semg1, sems0, sems1) = args
            y_hbm = None
        wid = lax.axis_index("s") * NC + lax.axis_index("c")
        base = pl.multiple_of(wid * RPW, RPW)
        jbase = pl.multiple_of(wid * JPW, JPW)
        bufs = (buf0, buf1)
        semgs = (semg0, semg1)
        semss = (sems0, sems1)

        pltpu.async_copy(idx_hbm, idx_v, semg0)
        pltpu.async_copy(w_hbm, w_v, semg1)
        pltpu.async_copy(valy_hbm, valy_v, sems0)
        pltpu.async_copy(idx3_hbm.at[wid], tgt_v, sems1)
        pltpu.make_async_copy(idx_hbm, idx_v, semg0).wait()
        pltpu.make_async_copy(w_hbm, w_v, semg1).wait()
        pltpu.make_async_copy(valy_hbm, valy_v, sems0).wait()
        pltpu.make_async_copy(idx3_hbm.at[wid], tgt_v, sems1).wait()

        nsc = JPW // K
        vws = []
        for c in range(nsc):
            off = pl.multiple_of(jbase + c * K, K)
            vt = idx_v[pl.ds(off, L)]
            vw = w_v[pl.ds(off, L)]
            tloc = jnp.clip(vt - lo, 0, HALF - 1)
            tgt2_v[c, :] = tloc
            vws.append(vw)
        pltpu.async_copy(val_hbm.at[vws[0]], buf0, semg0)
        for c in range(nsc):
            p = c % 2
            if c >= 1:
                pltpu.make_async_copy(bufs[1 - p], x_hbm.at[tgt2_v.at[c - 1]],
                                      semss[1 - p]).wait()
            if c + 1 < nsc:
                pltpu.async_copy(val_hbm.at[vws[c + 1]], bufs[1 - p],
                                 semgs[1 - p])
            pltpu.make_async_copy(val_hbm.at[vws[c]], bufs[p], semgs[p]).wait()
            pltpu.async_copy(bufs[p], x_hbm.at[tgt2_v.at[c]], semss[p])

        if with_y:
            pltpu.sync_copy(y_hbm.at[pl.ds(base, RPW)], myy_v)

            def _scan_body(c, carry):
                off = pl.multiple_of(c * L, L)
                vi = idx_v[pl.ds(off, L)]
                vw = w_v[pl.ds(off, L)]
                data = plsc.load_gather(valy_v, [vw])
                rel = vi - base
                inrange = (rel >= 0) & (rel < RPW)
                relc = jnp.clip(rel, 0, RPW - 1)
                plsc.store_scatter(myy_v, [relc], data, mask=inrange)
                return carry

            lax.fori_loop(0, B // L, _scan_body, jnp.int32(0))
            pltpu.sync_copy(myy_v, y_hbm.at[pl.ds(base, RPW)])

        pltpu.make_async_copy(bufs[(nsc - 1) % 2],
                              x_hbm.at[tgt2_v.at[nsc - 1]],
                              semss[(nsc - 1) % 2]).wait()
    return body


def _make_call(lo, with_y):
    mesh = plsc.VectorSubcoreMesh(core_axis_name="c", subcore_axis_name="s")
    return pl.kernel(
        _sc_body_half(lo, with_y),
        out_type=(),
        mesh=mesh,
        compiler_params=pltpu.CompilerParams(needs_layout_passes=False),
        scratch_types=[
            pltpu.VMEM((B,), jnp.int32),
            pltpu.VMEM((B,), jnp.int32),
            pltpu.VMEM((B,), jnp.int32),
            pltpu.VMEM((RPW,), jnp.int32),
            pltpu.VMEM((JPW // K, K), jnp.int32),
            pltpu.VMEM((JPW // K, K), jnp.int32),
            pltpu.VMEM((K, ROW), jnp.float32),
            pltpu.VMEM((K, ROW), jnp.float32),
            pltpu.SemaphoreType.DMA,
            pltpu.SemaphoreType.DMA,
            pltpu.SemaphoreType.DMA,
            pltpu.SemaphoreType.DMA,
        ],
    )


_WCHUNK = 512


def _winner_body(idx_col_ref, idx_row_ref, w_ref):
    col = idx_col_ref[...]
    row = idx_row_ref[...]
    match = col == row
    j = lax.broadcasted_iota(jnp.int32, (_WCHUNK, B), 1)
    w_ref[...] = jnp.max(jnp.where(match, j, -1), axis=1, keepdims=True)


def _compute_winners(idx):
    return pl.pallas_call(
        _winner_body,
        grid=(B // _WCHUNK,),
        in_specs=[
            pl.BlockSpec((_WCHUNK, 1), lambda c: (c, 0)),
            pl.BlockSpec((1, B), lambda c: (0, 0)),
        ],
        out_specs=pl.BlockSpec((_WCHUNK, 1), lambda c: (c, 0)),
        out_shape=jax.ShapeDtypeStruct((B, 1), jnp.int32),
    )(idx.reshape(B, 1), idx.reshape(1, B)).reshape(B)


def kernel(mem_x, mem_y, idx, val_x, val_y):
    w = _compute_winners(idx)
    val2 = val_x.reshape(B, ROW)
    idx3 = idx.reshape(NW, JPW // K, K)
    x0 = jax.new_ref(mem_x[:HALF].reshape(HALF, ROW))
    x1 = jax.new_ref(mem_x[HALF:].reshape(HALF, ROW))
    y_ref = jax.new_ref(mem_y)
    _make_call(0, True)(val2, idx, w, val_y, idx3, x0, y_ref)
    _make_call(HALF, False)(val2, idx, w, val_y, idx3, x1)
    out0 = x0[...].reshape(HALF, 3, 32, 32)
    out1 = x1[...].reshape(HALF, 3, 32, 32)
    return (jnp.concatenate([out0, out1], axis=0), y_ref[...])


# speedup vs baseline: 1.9906x; 1.9906x over previous
import jax
import jax.numpy as jnp
from jax import lax
from jax.experimental import pallas as pl
from jax.experimental.pallas import tpu as pltpu
from jax.experimental.pallas import tpu_sc as plsc

CAP = 16384
B = 4096
ROW = 3 * 32 * 32

NC, NS, L = 2, 16, 16
NW = NC * NS
RPW = CAP // NW
JPW = B // NW
K = 16

_WCHUNK = 512


def _winner_body(idx_col_ref, idx_row_ref, w_ref):
    col = idx_col_ref[...]
    row = idx_row_ref[...]
    match = col == row
    j = lax.broadcasted_iota(jnp.int32, (_WCHUNK, B), 1)
    w_ref[...] = jnp.max(jnp.where(match, j, -1), axis=1, keepdims=True)


def _compute_winners(idx):
    return pl.pallas_call(
        _winner_body,
        grid=(B // _WCHUNK,),
        in_specs=[
            pl.BlockSpec((_WCHUNK, 1), lambda c: (c, 0)),
            pl.BlockSpec((1, B), lambda c: (0, 0)),
        ],
        out_specs=pl.BlockSpec((_WCHUNK, 1), lambda c: (c, 0)),
        out_shape=jax.ShapeDtypeStruct((B, 1), jnp.int32),
    )(idx.reshape(B, 1), idx.reshape(1, B)).reshape(B)


def _sc_body(val_hbm, idx_hbm, w_hbm, valy_hbm, idx3_hbm, x_hbm, y_hbm,
             idx_v, w_v, valy_v, myy_v, tgt_v,
             buf0, buf1, semg0, semg1, sems0, sems1):
    wid = lax.axis_index("s") * NC + lax.axis_index("c")
    base = pl.multiple_of(wid * RPW, RPW)
    jbase = pl.multiple_of(wid * JPW, JPW)

    bufs = (buf0, buf1)
    semgs = (semg0, semg1)
    semss = (sems0, sems1)

    pltpu.async_copy(idx_hbm, idx_v, semg0)
    pltpu.async_copy(w_hbm, w_v, semg1)
    pltpu.async_copy(valy_hbm, valy_v, sems0)
    pltpu.async_copy(y_hbm.at[pl.ds(base, RPW)], myy_v, sems0)
    pltpu.async_copy(idx3_hbm.at[wid], tgt_v, sems1)
    pltpu.make_async_copy(idx_hbm, idx_v, semg0).wait()
    pltpu.make_async_copy(w_hbm, w_v, semg1).wait()
    pltpu.make_async_copy(valy_hbm, valy_v, sems0).wait()
    pltpu.make_async_copy(y_hbm.at[pl.ds(base, RPW)], myy_v, sems0).wait()
    pltpu.make_async_copy(idx3_hbm.at[wid], tgt_v, sems1).wait()

    nsc = JPW // K
    vws = [w_v[pl.ds(pl.multiple_of(jbase + c * K, K), K)] for c in range(nsc)]
    pltpu.async_copy(val_hbm.at[vws[0]], buf0, semg0)
    for c in range(nsc):
        p = c % 2
        if c >= 1:
            pltpu.make_async_copy(bufs[1 - p], x_hbm.at[tgt_v.at[c - 1]],
                                  semss[1 - p]).wait()
        if c + 1 < nsc:
            pltpu.async_copy(val_hbm.at[vws[c + 1]], bufs[1 - p],
                             semgs[1 - p])
        pltpu.make_async_copy(val_hbm.at[vws[c]], bufs[p], semgs[p]).wait()
        pltpu.async_copy(bufs[p], x_hbm.at[tgt_v.at[c]], semss[p])

    def _scan_body(c, carry):
        off = pl.multiple_of(c * L, L)
        vi = idx_v[pl.ds(off, L)]
        vw = w_v[pl.ds(off, L)]
        data = plsc.load_gather(valy_v, [vw])
        rel = vi - base
        inrange = (rel >= 0) & (rel < RPW)
        relc = jnp.clip(rel, 0, RPW - 1)
        plsc.store_scatter(myy_v, [relc], data, mask=inrange)
        return carry

    lax.fori_loop(0, B // L, _scan_body, jnp.int32(0))
    pltpu.sync_copy(myy_v, y_hbm.at[pl.ds(base, RPW)])

    pltpu.make_async_copy(bufs[(nsc - 1) % 2], x_hbm.at[tgt_v.at[nsc - 1]],
                          semss[(nsc - 1) % 2]).wait()


def _sc_scatter(x_ref, y_ref, val2, idx, w, val_y, idx3):
    mesh = plsc.VectorSubcoreMesh(core_axis_name="c", subcore_axis_name="s")
    f = pl.kernel(
        _sc_body,
        out_type=(),
        mesh=mesh,
        compiler_params=pltpu.CompilerParams(needs_layout_passes=False),
        scratch_types=[
            pltpu.VMEM((B,), jnp.int32),
            pltpu.VMEM((B,), jnp.int32),
            pltpu.VMEM((B,), jnp.int32),
            pltpu.VMEM((RPW,), jnp.int32),
            pltpu.VMEM((JPW // K, K), jnp.int32),
            pltpu.VMEM((K, ROW), jnp.float32),
            pltpu.VMEM((K, ROW), jnp.float32),
            pltpu.SemaphoreType.DMA,
            pltpu.SemaphoreType.DMA,
            pltpu.SemaphoreType.DMA,
            pltpu.SemaphoreType.DMA,
        ],
    )
    f(val2, idx, w, val_y, idx3, x_ref, y_ref)


def kernel(mem_x, mem_y, idx, val_x, val_y):
    w = _compute_winners(idx)
    mem2 = mem_x.reshape(CAP, ROW)
    val2 = val_x.reshape(B, ROW)
    idx3 = idx.reshape(NW, JPW // K, K)
    x_ref = jax.new_ref(mem2)
    y_ref = jax.new_ref(mem_y)
    _sc_scatter(x_ref, y_ref, val2, idx, w, val_y, idx3)
    return (x_ref[...].reshape(CAP, 3, 32, 32), y_ref[...])
